# baseline (speedup 1.0000x reference)
import functools

import jax
import jax.numpy as jnp
from jax import lax
from jax.experimental import pallas as pl
from jax.experimental.pallas import tpu as pltpu
from jax.experimental.pallas import tpu_sc as plsc

L = 16
W = 128
H = 128
NV = W // L
IMG = H * W
G = 144
CH = 64
NCH = H // CH
NCORES = 2
NSUB = 16
NWORKERS = NCORES * NSUB
NIMG = 2 * 96
PER_W = NIMG // NWORKERS
NTAP = 9

TAPS = [(di, dj) for di in (-1, 0, 1) for dj in (-1, 0, 1)]


def _body(s_hbm, cs_hbm, so_hbm, co_hbm, img, tb):
    cid = lax.axis_index("c")
    sid = lax.axis_index("s")
    wid = sid * NCORES + cid

    lanes = lax.iota(jnp.int32, L)
    zeros = jnp.zeros((L,), jnp.float32)
    ones = jnp.ones((L,), jnp.float32)
    m_first = jnp.where(lanes > 0, 1.0, 0.0).astype(jnp.float32)
    m_last = jnp.where(lanes < L - 1, 1.0, 0.0).astype(jnp.float32)

    for i in range(G // L):
        img[pl.ds(i * L, L)] = zeros
        img[pl.ds(G + IMG + i * L, L)] = zeros

    def do_rows(j, r0):
        h = r0 + j
        cbase = G + h * W
        cv = [img[pl.ds(cbase + L * v, L)] for v in range(NV)]
        rs_m = lax.broadcast(jnp.where(h - 1 >= 0, 1.0, 0.0), (L,))
        rs_p = lax.broadcast(jnp.where(h + 1 < H, 1.0, 0.0), (L,))
        slot = 0
        for t, (di, dj) in enumerate(TAPS):
            if t == 4:
                continue
            sbase = G + (h + di) * W + dj
            rs = rs_m if di == -1 else (rs_p if di == 1 else None)
            for v in range(NV):
                sv = img[pl.ds(sbase + L * v, L)]
                p = sv * cv[v]
                if rs is not None:
                    p = p * rs
                if dj == -1 and v == 0:
                    p = p * m_first
                if dj == 1 and v == NV - 1:
                    p = p * m_last
                tb[pl.ds(slot * CH * W + j * W + L * v, L)] = p
            slot += 1
        return r0

    def do_image(x_hbm, o_hbm, n):
        pltpu.sync_copy(x_hbm.at[n], img.at[pl.ds(G, IMG)])

        def do_chunk(c, n9):
            r0 = c * CH
            lax.fori_loop(0, CH, do_rows, r0)
            slot = 0
            for t in range(NTAP):
                if t == 4:
                    src = img.at[pl.ds(G + r0 * W, CH * W)]
                else:
                    src = tb.at[pl.ds(slot * CH * W, CH * W)]
                    slot += 1
                pltpu.sync_copy(src, o_hbm.at[n9 + t, pl.ds(r0 * W, CH * W)])
            return n9

        lax.fori_loop(0, NCH, do_chunk, n * NTAP)

    def s_loop(i, w):
        do_image(s_hbm, so_hbm, w * PER_W + i)
        return w

    def cs_loop(i, w):
        do_image(cs_hbm, co_hbm, w * PER_W + i)
        return w

    lax.fori_loop(0, PER_W, s_loop, wid)
    lax.fori_loop(0, PER_W, cs_loop, wid)


@functools.partial(jax.jit, donate_argnums=())
def _run(s2, cs2):
    mesh = plsc.VectorSubcoreMesh(
        core_axis_name="c", subcore_axis_name="s",
        num_cores=NCORES, num_subcores=NSUB,
    )
    out = jax.ShapeDtypeStruct((NIMG * NTAP, IMG), jnp.float32)
    return pl.kernel(
        _body,
        out_type=(out, out),
        mesh=mesh,
        scratch_types=[
            pltpu.VMEM((2 * G + IMG,), jnp.float32),
            pltpu.VMEM(((NTAP - 1) * CH * W,), jnp.float32),
        ],
        compiler_params=pltpu.CompilerParams(use_tc_tiling_on_sc=False),
    )(s2, cs2)


def kernel(s, cs):
    B, C = s.shape[0], s.shape[1]
    s2 = s.reshape(NIMG, IMG)
    cs2 = cs.reshape(NIMG, IMG)
    so, co = _run(s2, cs2)
    shape = (B, C, NTAP, H, W)
    return so.reshape(shape), co.reshape(shape)


# speedup vs baseline: 5.5887x; 5.5887x over previous
import functools

import jax
import jax.numpy as jnp
from jax import lax
from jax.experimental import pallas as pl
from jax.experimental.pallas import tpu as pltpu
from jax.experimental.pallas import tpu_sc as plsc

L = 16
W = 128
H = 128
NV = W // L
IMG = H * W
G = 144
CH = 32
NCH = H // CH
NCORES = 2
NSUB = 16
NWORKERS = NCORES * NSUB
NIMG = 2 * 96
PER_W = NIMG // NWORKERS
NTAP = 9

TAPS = [(di, dj) for di in (-1, 0, 1) for dj in (-1, 0, 1)]


def _body(s_hbm, cs_hbm, so_hbm, co_hbm, img, tba, tbb, sema, semb):
    cid = lax.axis_index("c")
    sid = lax.axis_index("s")
    wid = sid * NCORES + cid

    lanes = lax.iota(jnp.int32, L)
    zeros = jnp.zeros((L,), jnp.float32)
    m_first = jnp.where(lanes > 0, 1.0, 0.0).astype(jnp.float32)
    m_last = jnp.where(lanes < L - 1, 1.0, 0.0).astype(jnp.float32)

    for i in range(G // L):
        img[pl.ds(i * L, L)] = zeros
        img[pl.ds(G + IMG + i * L, L)] = zeros

    def make_rows(tb):
        def do_rows(j, r0):
            h = r0 + j
            cbase = G + h * W
            cv = [img[pl.ds(cbase + L * v, L)] for v in range(NV)]
            slot = 0
            for t, (di, dj) in enumerate(TAPS):
                if t == 4:
                    continue
                sbase = G + (h + di) * W + dj
                for v in range(NV):
                    sv = img[pl.ds(sbase + L * v, L)]
                    p = sv * cv[v]
                    if dj == -1 and v == 0:
                        p = p * m_first
                    if dj == 1 and v == NV - 1:
                        p = p * m_last
                    tb[pl.ds(slot * CH * W + j * W + L * v, L)] = p
                slot += 1
            return r0

        return do_rows

    rows_a = make_rows(tba)
    rows_b = make_rows(tbb)

    def fire(tb, sem, o_hbm, n9, c):
        r0 = c * CH
        handles = []
        slot = 0
        for t in range(NTAP):
            if t == 4:
                src = img.at[pl.ds(G + r0 * W, CH * W)]
            else:
                src = tb.at[pl.ds(slot * CH * W, CH * W)]
                slot += 1
            handles.append(
                pltpu.async_copy(src, o_hbm.at[n9 + t, pl.ds(r0 * W, CH * W)], sem)
            )
        return handles

    def do_image(x_hbm, o_hbm, n):
        pltpu.sync_copy(x_hbm.at[n], img.at[pl.ds(G, IMG)])
        n9 = n * NTAP
        lax.fori_loop(0, CH, rows_a, 0 * CH)
        ha0 = fire(tba, sema, o_hbm, n9, 0)
        lax.fori_loop(0, CH, rows_b, 1 * CH)
        hb1 = fire(tbb, semb, o_hbm, n9, 1)
        for hd in ha0:
            hd.wait()
        lax.fori_loop(0, CH, rows_a, 2 * CH)
        ha2 = fire(tba, sema, o_hbm, n9, 2)
        for hd in hb1:
            hd.wait()
        lax.fori_loop(0, CH, rows_b, 3 * CH)
        hb3 = fire(tbb, semb, o_hbm, n9, 3)
        for hd in ha2:
            hd.wait()
        for hd in hb3:
            hd.wait()

    def s_loop(i, w):
        do_image(s_hbm, so_hbm, w * PER_W + i)
        return w

    def cs_loop(i, w):
        do_image(cs_hbm, co_hbm, w * PER_W + i)
        return w

    lax.fori_loop(0, PER_W, s_loop, wid)
    lax.fori_loop(0, PER_W, cs_loop, wid)


@jax.jit
def _run(s2, cs2):
    mesh = plsc.VectorSubcoreMesh(
        core_axis_name="c", subcore_axis_name="s",
        num_cores=NCORES, num_subcores=NSUB,
    )
    out = jax.ShapeDtypeStruct((NIMG * NTAP, IMG), jnp.float32)
    return pl.kernel(
        _body,
        out_type=(out, out),
        mesh=mesh,
        scratch_types=[
            pltpu.VMEM((2 * G + IMG,), jnp.float32),
            pltpu.VMEM(((NTAP - 1) * CH * W,), jnp.float32),
            pltpu.VMEM(((NTAP - 1) * CH * W,), jnp.float32),
            pltpu.SemaphoreType.DMA,
            pltpu.SemaphoreType.DMA,
        ],
        compiler_params=pltpu.CompilerParams(use_tc_tiling_on_sc=False),
    )(s2, cs2)


def kernel(s, cs):
    B, C = s.shape[0], s.shape[1]
    s2 = s.reshape(NIMG, IMG)
    cs2 = cs.reshape(NIMG, IMG)
    so, co = _run(s2, cs2)
    shape = (B, C, NTAP, H, W)
    return so.reshape(shape), co.reshape(shape)
